# Initial kernel scaffold; baseline (speedup 1.0000x reference)
import jax
import jax.numpy as jnp
from jax.experimental import pallas as pl


def kernel(nfeat, edge_index, efeat, atom_table, edge_tables, W, b, gamma, beta, Wp, bp):
    raise NotImplementedError("write your pallas kernel here")


# speedup vs baseline: 5.4831x; 5.4831x over previous
import functools

import jax
import jax.numpy as jnp
from jax import lax
from jax.experimental import pallas as pl
from jax.experimental.pallas import tpu as pltpu
from jax.experimental.pallas import tpu_sc as plsc

N = 10000
E = 320000
D = 128
NBOND = 5
NATOM = 119

NCORES = 2
NSUBS = 16
NTILES = NCORES * NSUBS
B = 128
NSUB = -(-E // (NTILES * B))
EPAD = NTILES * NSUB * B

NACC = 10240
ZROWS = NACC // NSUBS

HL = 51200
HZ = HL // NSUBS

_mesh = plsc.VectorSubcoreMesh(core_axis_name="c", subcore_axis_name="s")


@functools.partial(
    pl.kernel,
    out_type=jax.ShapeDtypeStruct((NCORES, 1, HL), jnp.float32),
    mesh=_mesh,
    scratch_types=[
        pltpu.VMEM_SHARED((HL,), jnp.float32),
        pltpu.VMEM((NSUB, B), jnp.int32),
        pltpu.VMEM((NSUB, B), jnp.int32),
        pltpu.VMEM((B,), jnp.int32),
        pltpu.VMEM((B,), jnp.float32),
    ],
)
def _hist(dst_hbm, ef_hbm, zeros_hbm, out_hbm, acc, dst_v, ef_v, idx_v, ones_v):
    cid = lax.axis_index("c")
    sid = lax.axis_index("s")
    wid = sid * NCORES + cid
    pltpu.sync_copy(zeros_hbm, acc.at[pl.ds(sid * HZ, HZ)])
    pltpu.sync_copy(dst_hbm.at[wid], dst_v)
    pltpu.sync_copy(ef_hbm.at[wid], ef_v)
    for k in range(B // 16):
        ones_v[pl.ds(k * 16, 16)] = jnp.ones((16,), jnp.float32)
    plsc.subcore_barrier()

    def body(j, carry):
        for k in range(B // 16):
            d = dst_v[j, pl.ds(k * 16, 16)]
            e = ef_v[j, pl.ds(k * 16, 16)]
            idx_v[pl.ds(k * 16, 16)] = d * NBOND + e
        pltpu.sync_copy(ones_v, acc.at[idx_v], add=True)
        return carry

    lax.fori_loop(0, NSUB, body, 0)
    plsc.subcore_barrier()
    pltpu.sync_copy(acc.at[pl.ds(sid * HZ, HZ)],
                    out_hbm.at[cid, 0, pl.ds(sid * HZ, HZ)])


@functools.partial(
    pl.kernel,
    out_type=jax.ShapeDtypeStruct((NCORES, NACC, D), jnp.float32),
    mesh=_mesh,
    scratch_types=[
        pltpu.VMEM_SHARED((NACC, D), jnp.float32),
        pltpu.VMEM((NSUB, B), jnp.int32),
        pltpu.VMEM((NSUB, B), jnp.int32),
        pltpu.VMEM((B, D), jnp.float32),
    ],
)
def _segsum(h_hbm, src_hbm, dst_hbm, zeros_hbm, out_hbm, acc, src_v, dst_v, buf):
    cid = lax.axis_index("c")
    sid = lax.axis_index("s")
    wid = sid * NCORES + cid
    pltpu.sync_copy(zeros_hbm, acc.at[pl.ds(sid * ZROWS, ZROWS)])
    pltpu.sync_copy(src_hbm.at[wid], src_v)
    pltpu.sync_copy(dst_hbm.at[wid], dst_v)
    plsc.subcore_barrier()

    def body(j, carry):
        pltpu.sync_copy(h_hbm.at[src_v.at[j]], buf)
        pltpu.sync_copy(buf, acc.at[dst_v.at[j]], add=True)
        return carry

    lax.fori_loop(0, NSUB, body, 0)
    plsc.subcore_barrier()
    pltpu.sync_copy(acc.at[pl.ds(sid * ZROWS, ZROWS)],
                    out_hbm.at[cid, pl.ds(sid * ZROWS, ZROWS)])


def _h0_body(nf_ref, tab_ref, out_ref):
    nf = nf_ref[...]
    oh = (nf == lax.broadcasted_iota(jnp.int32, (N, NATOM), 1)).astype(jnp.float32)
    out_ref[...] = jnp.dot(oh, tab_ref[...], preferred_element_type=jnp.float32)


_h0 = pl.pallas_call(_h0_body, out_shape=jax.ShapeDtypeStruct((N, D), jnp.float32))


def _layer_body(final, h_ref, p_ref, c_ref, t_ref, w_ref, b_ref, g_ref, be_ref,
                wp_ref, bp_ref, out_ref):
    C = c_ref[0] + c_ref[1]
    degs = jnp.sum(C, axis=1, keepdims=True) + 1.0
    neigh = p_ref[0, :N] + p_ref[1, :N] + jnp.dot(C, t_ref[...],
                                                  preferred_element_type=jnp.float32)
    x = (h_ref[...] + neigh) / degs
    h2 = jnp.dot(x, w_ref[...], preferred_element_type=jnp.float32) + b_ref[...]
    mean = jnp.mean(h2, axis=0, keepdims=True)
    cent = h2 - mean
    var = jnp.mean(cent * cent, axis=0, keepdims=True)
    y = jnp.maximum(cent * lax.rsqrt(var + 1e-5) * g_ref[...] + be_ref[...], 0.0)
    if final:
        gm = jnp.mean(y, axis=0, keepdims=True)
        out_ref[...] = jnp.dot(gm, wp_ref[...],
                               preferred_element_type=jnp.float32) + bp_ref[...]
    else:
        out_ref[...] = y


_layer_mid = pl.pallas_call(functools.partial(_layer_body, False),
                            out_shape=jax.ShapeDtypeStruct((N, D), jnp.float32))
_layer_fin = pl.pallas_call(functools.partial(_layer_body, True),
                            out_shape=jax.ShapeDtypeStruct((1, 1), jnp.float32))


def kernel(nfeat, edge_index, efeat, atom_table, edge_tables, W, b, gamma, beta,
           Wp, bp):
    src = edge_index[0].astype(jnp.int32)
    dst = edge_index[1].astype(jnp.int32)
    ef = efeat.astype(jnp.int32)
    pad = EPAD - E
    srcr = jnp.pad(src, (0, pad)).reshape(NTILES, NSUB, B)
    dstr = jnp.pad(dst, (0, pad), constant_values=N).reshape(NTILES, NSUB, B)
    efr = jnp.pad(ef, (0, pad)).reshape(NTILES, NSUB, B)

    hzeros = jnp.zeros((HZ,), jnp.float32)
    szeros = jnp.zeros((ZROWS, D), jnp.float32)

    hist = _hist(dstr, efr, hzeros)
    histc = hist[:, 0, :NBOND * N].reshape(NCORES, N, NBOND)

    h = _h0(nfeat.astype(jnp.int32).reshape(N, 1), atom_table)
    out = None
    for l in range(3):
        parts = _segsum(h, srcr, dstr, szeros)
        args = (h, parts, histc, edge_tables[l], W[l], b[l].reshape(1, D),
                gamma[l].reshape(1, D), beta[l].reshape(1, D),
                Wp, bp.reshape(1, 1))
        if l < 2:
            h = _layer_mid(*args)
        else:
            out = _layer_fin(*args)
    return out
